# Initial kernel scaffold; baseline (speedup 1.0000x reference)
import jax
import jax.numpy as jnp
from jax.experimental import pallas as pl


def kernel(z_g, weight):
    raise NotImplementedError("write your pallas kernel here")


# speedup vs baseline: 2.1517x; 2.1517x over previous
import functools

import jax
import jax.numpy as jnp
from jax.experimental import pallas as pl


def _vq_body(z_ref, w_ref, out_ref, *, K):
    z = z_ref[0]
    w = w_ref[...]
    wsq = jnp.sum(w * w, axis=0, keepdims=True)
    scores = jax.lax.dot_general(
        z, w, (((0,), (0,)), ((), ())),
        preferred_element_type=jnp.float32)
    zsq = jax.lax.dot_general(
        z * z, jnp.ones((z.shape[0], 1), jnp.float32),
        (((0,), (0,)), ((), ())),
        preferred_element_type=jnp.float32)
    dist = (zsq - 2.0 * scores) + wsq
    mind = jnp.min(dist, axis=1, keepdims=True)
    iota = jax.lax.broadcasted_iota(jnp.int32, dist.shape, 1)
    cand = jnp.where(dist == mind, iota, K)
    idx = jnp.min(cand, axis=1, keepdims=True)
    onehot = (iota == idx).astype(jnp.float32)
    q = jax.lax.dot_general(
        w, onehot, (((1,), (1,)), ((), ())),
        preferred_element_type=jnp.float32)
    out_ref[0] = q


def kernel(z_g, weight):
    B, D, H, W = z_g.shape
    K = weight.shape[1]
    HW = H * W
    z3 = z_g.reshape(B, D, HW)
    q3 = pl.pallas_call(
        functools.partial(_vq_body, K=K),
        grid=(B,),
        in_specs=[
            pl.BlockSpec((1, D, HW), lambda i: (i, 0, 0)),
            pl.BlockSpec((D, K), lambda i: (0, 0)),
        ],
        out_specs=pl.BlockSpec((1, D, HW), lambda i: (i, 0, 0)),
        out_shape=jax.ShapeDtypeStruct((B, D, HW), jnp.float32),
    )(z3, weight)
    q = q3.reshape(B, D, H, W)
    return (q, q)


# speedup vs baseline: 2.4574x; 1.1421x over previous
import functools

import jax
import jax.numpy as jnp
from jax.experimental import pallas as pl


def _vq_body(z_ref, w_ref, zq_ref, emb_ref, *, K):
    z = z_ref[0]
    w = w_ref[...]
    D = w.shape[0]
    HW = z.shape[1]
    ones_d1 = jnp.ones((D, 1), jnp.float32)
    wsq = jax.lax.dot_general(
        w * w, ones_d1, (((0,), (0,)), ((), ())),
        preferred_element_type=jnp.float32)
    zsq = jax.lax.dot_general(
        ones_d1, z * z, (((0,), (0,)), ((), ())),
        preferred_element_type=jnp.float32)
    scores = jax.lax.dot_general(
        w, z, (((0,), (0,)), ((), ())),
        preferred_element_type=jnp.float32)
    dist = (zsq - 2.0 * scores) + wsq
    mind = jnp.min(dist, axis=0, keepdims=True)
    iota = jax.lax.broadcasted_iota(jnp.int32, dist.shape, 0)
    cand = jnp.where(dist == mind, iota, K)
    idx = jnp.min(cand, axis=0, keepdims=True)
    onehot = (iota == idx).astype(jnp.float32)
    q = jax.lax.dot_general(
        w, onehot, (((1,), (0,)), ((), ())),
        preferred_element_type=jnp.float32)
    zq_ref[0] = q
    emb_ref[0] = q


def kernel(z_g, weight):
    B, D, H, W = z_g.shape
    K = weight.shape[1]
    HW = H * W
    z3 = z_g.reshape(B, D, HW)
    out_sds = jax.ShapeDtypeStruct((B, D, HW), jnp.float32)
    zq3, emb3 = pl.pallas_call(
        functools.partial(_vq_body, K=K),
        grid=(B,),
        in_specs=[
            pl.BlockSpec((1, D, HW), lambda i: (i, 0, 0)),
            pl.BlockSpec((D, K), lambda i: (0, 0)),
        ],
        out_specs=[
            pl.BlockSpec((1, D, HW), lambda i: (i, 0, 0)),
            pl.BlockSpec((1, D, HW), lambda i: (i, 0, 0)),
        ],
        out_shape=[out_sds, out_sds],
    )(z3, weight)
    return (zq3.reshape(B, D, H, W), emb3.reshape(B, D, H, W))
